# Initial kernel scaffold; baseline (speedup 1.0000x reference)
import jax
import jax.numpy as jnp
from jax.experimental import pallas as pl


def kernel(x, edge_index, W1, b1, gamma, beta, attn_W, attn_b, fc_W, fc_b):
    raise NotImplementedError("write your pallas kernel here")


# speedup vs baseline: 19.5321x; 19.5321x over previous
import jax
import jax.numpy as jnp
from jax import lax
from jax.experimental import pallas as pl
from jax.experimental.pallas import tpu as pltpu
from jax.experimental.pallas import tpu_sc as plsc

N = 10000
F_IN = 128
HID = 256
C_OUT = 64
E = 320000

NC, NS = 2, 16
NW = NC * NS
EP = E // NW
CH = 80
NCHUNK = EP // CH
CH1 = 400
NACC = 10240
SLC = NACC // NS

_MESH = plsc.VectorSubcoreMesh(
    core_axis_name="c", subcore_axis_name="s", num_cores=NC, num_subcores=NS)


def _deg_body(dst_hbm, zeros_hbm, out_hbm, deg_v, dstbuf_v):
    c = lax.axis_index("c")
    s = lax.axis_index("s")
    wid = c * NS + s
    pltpu.sync_copy(zeros_hbm, deg_v)
    ones = jnp.full((16,), 1.0, jnp.float32)

    def chunk(i, carry):
        base = wid * EP + i * CH1
        pltpu.sync_copy(dst_hbm.at[pl.ds(base, CH1)], dstbuf_v)

        def inner(j, carry2):
            idx = dstbuf_v[pl.ds(j * 16, 16)]
            plsc.addupdate_scatter(deg_v, [idx], ones)
            return carry2

        return lax.fori_loop(0, CH1 // 16, inner, carry)

    lax.fori_loop(0, EP // CH1, chunk, 0)
    pltpu.sync_copy(deg_v, out_hbm.at[wid])


def _deg_partials(dst):
    zeros1d = jnp.zeros((NACC,), jnp.float32)
    return pl.kernel(
        _deg_body,
        out_type=jax.ShapeDtypeStruct((NW, NACC), jnp.float32),
        mesh=_MESH,
        scratch_types=[
            pltpu.VMEM((NACC,), jnp.float32),
            pltpu.VMEM((CH1,), jnp.int32),
        ],
        compiler_params=pltpu.CompilerParams(needs_layout_passes=False),
    )(dst, zeros1d)


def _segsum_body(xs_hbm, src_hbm, dst_hbm, zrows_hbm, out_hbm,
                 src_v, dst_v, rows_v, acc, sem):
    c = lax.axis_index("c")
    s = lax.axis_index("s")
    wid = c * NS + s
    pltpu.sync_copy(zrows_hbm, acc.at[pl.ds(s * SLC, SLC)])
    plsc.subcore_barrier()

    def chunk(i, carry):
        base = wid * EP + i * CH
        pltpu.sync_copy(src_hbm.at[pl.ds(base, CH)], src_v)
        pltpu.sync_copy(dst_hbm.at[pl.ds(base, CH)], dst_v)
        pltpu.async_copy(xs_hbm.at[src_v], rows_v, sem).wait()
        pltpu.sync_copy(rows_v, acc.at[dst_v], add=True)
        return carry

    lax.fori_loop(0, NCHUNK, chunk, 0)
    plsc.subcore_barrier()
    off = c * NACC + s * SLC
    pltpu.sync_copy(acc.at[pl.ds(s * SLC, SLC)],
                    out_hbm.at[pl.ds(off, SLC)])


def _segsum(xs, src, dst):
    zrows = jnp.zeros((SLC, F_IN), jnp.float32)
    flat = pl.kernel(
        _segsum_body,
        out_type=jax.ShapeDtypeStruct((NC * NACC, F_IN), jnp.float32),
        mesh=_MESH,
        scratch_types=[
            pltpu.VMEM((CH,), jnp.int32),
            pltpu.VMEM((CH,), jnp.int32),
            pltpu.VMEM((CH, F_IN), jnp.float32),
            pltpu.VMEM_SHARED((NACC, F_IN), jnp.float32),
            pltpu.SemaphoreType.DMA,
        ],
    )(xs, src, dst, zrows)
    return flat.reshape(NC, NACC, F_IN)


def _scale_body(x_ref, degp_ref, xs_ref, dis_ref):
    deg = jnp.sum(degp_ref[...], axis=1, keepdims=True) + 1.0
    dis = lax.rsqrt(deg)
    xs_ref[...] = x_ref[...] * dis
    dis_ref[...] = dis


def _scale(x, deg_part_t, blk=512):
    grid = (pl.cdiv(N, blk),)
    return pl.pallas_call(
        _scale_body,
        grid=grid,
        in_specs=[
            pl.BlockSpec((blk, F_IN), lambda i: (i, 0)),
            pl.BlockSpec((blk, NW), lambda i: (i, 0)),
        ],
        out_specs=[
            pl.BlockSpec((blk, F_IN), lambda i: (i, 0)),
            pl.BlockSpec((blk, 1), lambda i: (i, 0)),
        ],
        out_shape=[
            jax.ShapeDtypeStruct((N, F_IN), jnp.float32),
            jax.ShapeDtypeStruct((N, 1), jnp.float32),
        ],
    )(x, deg_part_t)


def _post_body(p_ref, xs_ref, dis_ref, x_ref, w1_ref, b1_ref, gamma_ref,
               beta_ref, aw_ref, ab_ref, fw_ref, fb_ref, out_ref):
    pr = p_ref[...]
    xa = (pr[0] + pr[1] + xs_ref[...]) * dis_ref[...]
    agg = jnp.dot(xa, w1_ref[...], preferred_element_type=jnp.float32)
    agg = agg + b1_ref[...]
    mean = jnp.mean(agg, axis=1, keepdims=True)
    cent = agg - mean
    var = jnp.mean(cent * cent, axis=1, keepdims=True)
    hn = cent * lax.rsqrt(var + 1e-5) * gamma_ref[...] + beta_ref[...]
    hr = jnp.maximum(hn, 0.0)
    comb = jnp.concatenate([hr, x_ref[...]], axis=1)
    att = jax.nn.sigmoid(
        jnp.dot(comb, aw_ref[...], preferred_element_type=jnp.float32)
        + ab_ref[...])
    out_ref[...] = (
        jnp.dot(comb * att, fw_ref[...], preferred_element_type=jnp.float32)
        + fb_ref[...])


def _post(p, xs, dis, x, W1, b1, gamma, beta, attn_W, attn_b,
          fc_W, fc_b, blk=512):
    grid = (pl.cdiv(N, blk),)
    full = lambda i: (0, 0)
    return pl.pallas_call(
        _post_body,
        grid=grid,
        in_specs=[
            pl.BlockSpec((NC, blk, F_IN), lambda i: (0, i, 0)),
            pl.BlockSpec((blk, F_IN), lambda i: (i, 0)),
            pl.BlockSpec((blk, 1), lambda i: (i, 0)),
            pl.BlockSpec((blk, F_IN), lambda i: (i, 0)),
            pl.BlockSpec((F_IN, HID), full),
            pl.BlockSpec((1, HID), full),
            pl.BlockSpec((1, HID), full),
            pl.BlockSpec((1, HID), full),
            pl.BlockSpec((HID + F_IN, 1), full),
            pl.BlockSpec((1, 1), full),
            pl.BlockSpec((HID + F_IN, C_OUT), full),
            pl.BlockSpec((1, C_OUT), full),
        ],
        out_specs=pl.BlockSpec((blk, C_OUT), lambda i: (i, 0)),
        out_shape=jax.ShapeDtypeStruct((N, C_OUT), jnp.float32),
    )(p, xs, dis, x, W1, b1, gamma, beta, attn_W, attn_b, fc_W, fc_b)


def kernel(x, edge_index, W1, b1, gamma, beta, attn_W, attn_b, fc_W, fc_b):
    edge_index = edge_index.astype(jnp.int32)
    src = edge_index[0]
    dst = edge_index[1]

    deg_part = _deg_partials(dst)
    xs, dis = _scale(x, deg_part.T)
    p = _segsum(xs, src, dst)
    return _post(
        p, xs, dis, x, W1,
        b1.reshape(1, HID), gamma.reshape(1, HID), beta.reshape(1, HID),
        attn_W, attn_b.reshape(1, 1), fc_W, fc_b.reshape(1, C_OUT))


# speedup vs baseline: 39.9227x; 2.0440x over previous
import jax
import jax.numpy as jnp
from jax import lax
from jax.experimental import pallas as pl
from jax.experimental.pallas import tpu as pltpu
from jax.experimental.pallas import tpu_sc as plsc

N = 10000
F_IN = 128
HID = 256
C_OUT = 64
E = 320000

NC, NS = 2, 16
NW = NC * NS
NACC = 10240
SLC = NACC // NS

EPP = 10240
EPAD = EPP * NW
CH = 128
NCH = EPP // CH
CH1 = 512

_MESH = plsc.VectorSubcoreMesh(
    core_axis_name="c", subcore_axis_name="s", num_cores=NC, num_subcores=NS)


EP = E // NW


def _deg_body(dst_hbm, zeros_hbm, out_hbm, deg_v, dstbuf_v):
    c = lax.axis_index("c")
    s = lax.axis_index("s")
    wid = c * NS + s
    pltpu.sync_copy(zeros_hbm, deg_v)
    pltpu.sync_copy(dst_hbm.at[pl.ds(wid * EP, EP)], dstbuf_v)
    ones = jnp.full((16,), 1.0, jnp.float32)

    def inner(j, carry):
        idx = dstbuf_v[pl.ds(j * 16, 16)]
        plsc.addupdate_scatter(deg_v, [idx], ones)
        return carry

    lax.fori_loop(0, EP // 16, inner, 0, unroll=5)
    pltpu.sync_copy(deg_v, out_hbm.at[wid])


def _deg_partials(dst):
    zeros1d = jnp.zeros((NACC,), jnp.float32)
    return pl.kernel(
        _deg_body,
        out_type=jax.ShapeDtypeStruct((NW, NACC), jnp.float32),
        mesh=_MESH,
        scratch_types=[
            pltpu.VMEM((NACC,), jnp.float32),
            pltpu.VMEM((EP,), jnp.int32),
        ],
        compiler_params=pltpu.CompilerParams(needs_layout_passes=False),
    )(dst, zeros1d)


def _segsum_body(xs_hbm, ei_hbm, zrows_hbm, out_hbm,
                 i0, i1, i2, i3, r0, r1, acc,
                 si0, si1, si2, si3, sr0, sr1):
    isl = (i0, i1, i2, i3)
    isem = (si0, si1, si2, si3)
    rsl = (r0, r1)
    rsem = (sr0, sr1)
    c = lax.axis_index("c")
    s = lax.axis_index("s")
    wid = c * NS + s
    ebase = wid * EPP

    def lstart(j, k4):
        pltpu.async_copy(
            ei_hbm.at[:, pl.ds(ebase + j * CH, CH)], isl[k4], isem[k4])

    def lwait(j, k4):
        pltpu.make_async_copy(
            ei_hbm.at[:, pl.ds(ebase + j * CH, CH)], isl[k4], isem[k4]).wait()

    def gstart(k4, k2):
        pltpu.async_copy(xs_hbm.at[isl[k4].at[0]], rsl[k2], rsem[k2])

    def gwait(k4, k2):
        pltpu.make_async_copy(
            xs_hbm.at[isl[k4].at[0]], rsl[k2], rsem[k2]).wait()

    def scat(k4, k2):
        pltpu.sync_copy(rsl[k2], acc.at[isl[k4].at[1]], add=True)

    pltpu.sync_copy(zrows_hbm, acc.at[pl.ds(s * SLC, SLC)])
    plsc.subcore_barrier()

    for j in range(4):
        lstart(j, j)
    lwait(0, 0)
    gstart(0, 0)

    M = NCH // 4 - 1

    def main(k, carry):
        j0 = 4 * k
        for b in range(4):
            jb = j0 + b
            lwait(jb + 1, (b + 1) % 4)
            gstart((b + 1) % 4, (b + 1) % 2)
            gwait(b, b % 2)
            scat(b, b % 2)
            lstart(jb + 4, b)
        return carry

    lax.fori_loop(0, M, main, 0)

    for b in range(4):
        jb = 4 * M + b
        if jb + 1 < NCH:
            lwait(jb + 1, (b + 1) % 4)
            gstart((b + 1) % 4, (b + 1) % 2)
        gwait(b, b % 2)
        scat(b, b % 2)

    plsc.subcore_barrier()
    off = c * NACC + s * SLC
    pltpu.sync_copy(acc.at[pl.ds(s * SLC, SLC)],
                    out_hbm.at[pl.ds(off, SLC)])


def _segsum(xs, ei):
    zrows = jnp.zeros((SLC, F_IN), jnp.float32)
    flat = pl.kernel(
        _segsum_body,
        out_type=jax.ShapeDtypeStruct((NC * NACC, F_IN), jnp.float32),
        mesh=_MESH,
        scratch_types=[
            pltpu.VMEM((2, CH), jnp.int32),
            pltpu.VMEM((2, CH), jnp.int32),
            pltpu.VMEM((2, CH), jnp.int32),
            pltpu.VMEM((2, CH), jnp.int32),
            pltpu.VMEM((CH, F_IN), jnp.float32),
            pltpu.VMEM((CH, F_IN), jnp.float32),
            pltpu.VMEM_SHARED((NACC, F_IN), jnp.float32),
            pltpu.SemaphoreType.DMA,
            pltpu.SemaphoreType.DMA,
            pltpu.SemaphoreType.DMA,
            pltpu.SemaphoreType.DMA,
            pltpu.SemaphoreType.DMA,
            pltpu.SemaphoreType.DMA,
        ],
    )(xs, ei, zrows)
    return flat.reshape(NC, NACC, F_IN)


def _scale_body(x_ref, degp_ref, xs_ref, dis_ref):
    deg = jnp.sum(degp_ref[...], axis=1, keepdims=True) + 1.0
    dis = lax.rsqrt(deg)
    xs_ref[...] = x_ref[...] * dis
    dis_ref[...] = dis


def _scale(x, deg_part_t, blk=512):
    grid = (pl.cdiv(N, blk),)
    return pl.pallas_call(
        _scale_body,
        grid=grid,
        in_specs=[
            pl.BlockSpec((blk, F_IN), lambda i: (i, 0)),
            pl.BlockSpec((blk, NW), lambda i: (i, 0)),
        ],
        out_specs=[
            pl.BlockSpec((blk, F_IN), lambda i: (i, 0)),
            pl.BlockSpec((blk, 1), lambda i: (i, 0)),
        ],
        out_shape=[
            jax.ShapeDtypeStruct((N, F_IN), jnp.float32),
            jax.ShapeDtypeStruct((N, 1), jnp.float32),
        ],
    )(x, deg_part_t)


def _post_body(p_ref, xs_ref, dis_ref, x_ref, w1_ref, b1_ref, gamma_ref,
               beta_ref, aw_ref, ab_ref, fw_ref, fb_ref, out_ref):
    pr = p_ref[...]
    xa = (pr[0] + pr[1] + xs_ref[...]) * dis_ref[...]
    agg = jnp.dot(xa, w1_ref[...], preferred_element_type=jnp.float32)
    agg = agg + b1_ref[...]
    mean = jnp.mean(agg, axis=1, keepdims=True)
    cent = agg - mean
    var = jnp.mean(cent * cent, axis=1, keepdims=True)
    hn = cent * lax.rsqrt(var + 1e-5) * gamma_ref[...] + beta_ref[...]
    hr = jnp.maximum(hn, 0.0)
    comb = jnp.concatenate([hr, x_ref[...]], axis=1)
    att = jax.nn.sigmoid(
        jnp.dot(comb, aw_ref[...], preferred_element_type=jnp.float32)
        + ab_ref[...])
    out_ref[...] = (
        jnp.dot(comb * att, fw_ref[...], preferred_element_type=jnp.float32)
        + fb_ref[...])


def _post(p, xs, dis, x, W1, b1, gamma, beta, attn_W, attn_b,
          fc_W, fc_b, blk=512):
    grid = (pl.cdiv(N, blk),)
    full = lambda i: (0, 0)
    return pl.pallas_call(
        _post_body,
        grid=grid,
        in_specs=[
            pl.BlockSpec((NC, blk, F_IN), lambda i: (0, i, 0)),
            pl.BlockSpec((blk, F_IN), lambda i: (i, 0)),
            pl.BlockSpec((blk, 1), lambda i: (i, 0)),
            pl.BlockSpec((blk, F_IN), lambda i: (i, 0)),
            pl.BlockSpec((F_IN, HID), full),
            pl.BlockSpec((1, HID), full),
            pl.BlockSpec((1, HID), full),
            pl.BlockSpec((1, HID), full),
            pl.BlockSpec((HID + F_IN, 1), full),
            pl.BlockSpec((1, 1), full),
            pl.BlockSpec((HID + F_IN, C_OUT), full),
            pl.BlockSpec((1, C_OUT), full),
        ],
        out_specs=pl.BlockSpec((blk, C_OUT), lambda i: (i, 0)),
        out_shape=jax.ShapeDtypeStruct((N, C_OUT), jnp.float32),
    )(p, xs, dis, x, W1, b1, gamma, beta, attn_W, attn_b, fc_W, fc_b)


def kernel(x, edge_index, W1, b1, gamma, beta, attn_W, attn_b, fc_W, fc_b):
    ei = edge_index.astype(jnp.int32)
    npad = EPAD - ei.shape[1]
    ar = jnp.arange(npad, dtype=jnp.int32)
    pad = jnp.stack([ar % N, N + ar % (NACC - N)])
    ei_p = jnp.concatenate([ei, pad], axis=1)

    deg_part = _deg_partials(ei[1])
    xs, dis = _scale(x, deg_part.T)
    p = _segsum(xs, ei_p)
    return _post(
        p, xs, dis, x, W1,
        b1.reshape(1, HID), gamma.reshape(1, HID), beta.reshape(1, HID),
        attn_W, attn_b.reshape(1, 1), fc_W, fc_b.reshape(1, C_OUT))


# speedup vs baseline: 44.1332x; 1.1055x over previous
import numpy as np

import jax
import jax.numpy as jnp
from jax import lax
from jax.experimental import pallas as pl
from jax.experimental.pallas import tpu as pltpu
from jax.experimental.pallas import tpu_sc as plsc

N = 10000
F_IN = 128
HID = 256
C_OUT = 64
E = 320000

NC, NS = 2, 16
NW = NC * NS
NACC = 10240
SLC = NACC // NS

EPP = 10240
EPAD = EPP * NW
CH = 128
NCH = EPP // CH
CH1 = 512

_MESH = plsc.VectorSubcoreMesh(
    core_axis_name="c", subcore_axis_name="s", num_cores=NC, num_subcores=NS)


DR = 128
DRS = DR // NS


def _deg_body(ei_hbm, zeros_hbm, iota_hbm, out_hbm,
              dstbuf_v, deg_v, idx80_v, accd):
    c = lax.axis_index("c")
    s = lax.axis_index("s")
    wid = c * NS + s
    pltpu.sync_copy(zeros_hbm, deg_v)
    pltpu.sync_copy(zeros_hbm.at[pl.ds(s * DRS, DRS)],
                    accd.at[pl.ds(s * DRS, DRS)])
    pltpu.sync_copy(iota_hbm, idx80_v)
    pltpu.sync_copy(ei_hbm.at[pl.ds(1, 1), pl.ds(wid * EPP, EPP)], dstbuf_v)
    ones = jnp.full((16,), 1.0, jnp.float32)

    def inner(j, carry):
        idx = dstbuf_v[0, pl.ds(j * 16, 16)]
        plsc.addupdate_scatter(
            deg_v,
            [lax.shift_right_logical(idx, 7), jnp.bitwise_and(idx, 127)],
            ones)
        return carry

    lax.fori_loop(0, EPP // 16, inner, 0, unroll=5)
    plsc.subcore_barrier()
    pltpu.sync_copy(deg_v, accd.at[idx80_v], add=True)
    plsc.subcore_barrier()
    pltpu.sync_copy(accd.at[pl.ds(s * DRS, DRS)],
                    out_hbm.at[pl.ds(c * DR + s * DRS, DRS)])


def _deg_partials(ei_p):
    zeros2d = jnp.zeros((DR, 128), jnp.float32)
    iota80 = jnp.arange(DR, dtype=jnp.int32)
    flat = pl.kernel(
        _deg_body,
        out_type=jax.ShapeDtypeStruct((NC * DR, 128), jnp.float32),
        mesh=_MESH,
        scratch_types=[
            pltpu.VMEM((1, EPP), jnp.int32),
            pltpu.VMEM((DR, 128), jnp.float32),
            pltpu.VMEM((DR,), jnp.int32),
            pltpu.VMEM_SHARED((DR, 128), jnp.float32),
        ],
        compiler_params=pltpu.CompilerParams(needs_layout_passes=False),
    )(ei_p, zeros2d, iota80)
    return flat


def _segsum_body(xs_hbm, ei_hbm, zrows_hbm, out_hbm,
                 i0, i1, i2, i3, r0, r1, acc,
                 si0, si1, si2, si3, sr0, sr1):
    isl = (i0, i1, i2, i3)
    isem = (si0, si1, si2, si3)
    rsl = (r0, r1)
    rsem = (sr0, sr1)
    c = lax.axis_index("c")
    s = lax.axis_index("s")
    wid = c * NS + s
    ebase = wid * EPP

    def lstart(j, k4):
        pltpu.async_copy(
            ei_hbm.at[:, pl.ds(ebase + j * CH, CH)], isl[k4], isem[k4])

    def lwait(j, k4):
        pltpu.make_async_copy(
            ei_hbm.at[:, pl.ds(ebase + j * CH, CH)], isl[k4], isem[k4]).wait()

    def gstart(k4, k2):
        pltpu.async_copy(xs_hbm.at[isl[k4].at[0]], rsl[k2], rsem[k2])

    def gwait(k4, k2):
        pltpu.make_async_copy(
            xs_hbm.at[isl[k4].at[0]], rsl[k2], rsem[k2]).wait()

    def scat(k4, k2):
        pltpu.sync_copy(rsl[k2], acc.at[isl[k4].at[1]], add=True)

    pltpu.sync_copy(zrows_hbm, acc.at[pl.ds(s * SLC, SLC)])
    plsc.subcore_barrier()

    for j in range(4):
        lstart(j, j)
    lwait(0, 0)
    gstart(0, 0)

    M = NCH // 4 - 1

    def main(k, carry):
        j0 = 4 * k
        for b in range(4):
            jb = j0 + b
            lwait(jb + 1, (b + 1) % 4)
            gstart((b + 1) % 4, (b + 1) % 2)
            gwait(b, b % 2)
            scat(b, b % 2)
            lstart(jb + 4, b)
        return carry

    lax.fori_loop(0, M, main, 0)

    for b in range(4):
        jb = 4 * M + b
        if jb + 1 < NCH:
            lwait(jb + 1, (b + 1) % 4)
            gstart((b + 1) % 4, (b + 1) % 2)
        gwait(b, b % 2)
        scat(b, b % 2)

    plsc.subcore_barrier()
    off = c * NACC + s * SLC
    pltpu.sync_copy(acc.at[pl.ds(s * SLC, SLC)],
                    out_hbm.at[pl.ds(off, SLC)])


def _segsum(xs, ei):
    zrows = jnp.zeros((SLC, F_IN), jnp.float32)
    flat = pl.kernel(
        _segsum_body,
        out_type=jax.ShapeDtypeStruct((NC * NACC, F_IN), jnp.float32),
        mesh=_MESH,
        scratch_types=[
            pltpu.VMEM((2, CH), jnp.int32),
            pltpu.VMEM((2, CH), jnp.int32),
            pltpu.VMEM((2, CH), jnp.int32),
            pltpu.VMEM((2, CH), jnp.int32),
            pltpu.VMEM((CH, F_IN), jnp.float32),
            pltpu.VMEM((CH, F_IN), jnp.float32),
            pltpu.VMEM_SHARED((NACC, F_IN), jnp.float32),
            pltpu.SemaphoreType.DMA,
            pltpu.SemaphoreType.DMA,
            pltpu.SemaphoreType.DMA,
            pltpu.SemaphoreType.DMA,
            pltpu.SemaphoreType.DMA,
            pltpu.SemaphoreType.DMA,
        ],
    )(xs, ei, zrows)
    return flat.reshape(NC, NACC, F_IN)


def _scale_body(x_ref, deg_ref, xs_ref, dis_ref):
    dis = lax.rsqrt(deg_ref[...] + 1.0)
    xs_ref[...] = x_ref[...] * dis
    dis_ref[...] = dis


def _scale(x, deg, blk=2000):
    grid = (pl.cdiv(N, blk),)
    return pl.pallas_call(
        _scale_body,
        grid=grid,
        in_specs=[
            pl.BlockSpec((blk, F_IN), lambda i: (i, 0)),
            pl.BlockSpec((blk, 1), lambda i: (i, 0)),
        ],
        out_specs=[
            pl.BlockSpec((blk, F_IN), lambda i: (i, 0)),
            pl.BlockSpec((blk, 1), lambda i: (i, 0)),
        ],
        out_shape=[
            jax.ShapeDtypeStruct((N, F_IN), jnp.float32),
            jax.ShapeDtypeStruct((N, 1), jnp.float32),
        ],
    )(x, deg)


def _post_body(p_ref, xs_ref, dis_ref, x_ref, w1_ref, b1_ref, gamma_ref,
               beta_ref, aw_ref, ab_ref, fw_ref, fb_ref, out_ref):
    pr = p_ref[...]
    xa = (pr[0] + pr[1] + xs_ref[...]) * dis_ref[...]
    agg = jnp.dot(xa, w1_ref[...], preferred_element_type=jnp.float32)
    agg = agg + b1_ref[...]
    mean = jnp.mean(agg, axis=1, keepdims=True)
    cent = agg - mean
    var = jnp.mean(cent * cent, axis=1, keepdims=True)
    hn = cent * lax.rsqrt(var + 1e-5) * gamma_ref[...] + beta_ref[...]
    hr = jnp.maximum(hn, 0.0)
    comb = jnp.concatenate([hr, x_ref[...]], axis=1)
    att = jax.nn.sigmoid(
        jnp.dot(comb, aw_ref[...], preferred_element_type=jnp.float32)
        + ab_ref[...])
    out_ref[...] = (
        jnp.dot(comb * att, fw_ref[...], preferred_element_type=jnp.float32)
        + fb_ref[...])


def _post(p, xs, dis, x, W1, b1, gamma, beta, attn_W, attn_b,
          fc_W, fc_b, blk=1000):
    grid = (pl.cdiv(N, blk),)
    full = lambda i: (0, 0)
    return pl.pallas_call(
        _post_body,
        grid=grid,
        in_specs=[
            pl.BlockSpec((NC, blk, F_IN), lambda i: (0, i, 0)),
            pl.BlockSpec((blk, F_IN), lambda i: (i, 0)),
            pl.BlockSpec((blk, 1), lambda i: (i, 0)),
            pl.BlockSpec((blk, F_IN), lambda i: (i, 0)),
            pl.BlockSpec((F_IN, HID), full),
            pl.BlockSpec((1, HID), full),
            pl.BlockSpec((1, HID), full),
            pl.BlockSpec((1, HID), full),
            pl.BlockSpec((HID + F_IN, 1), full),
            pl.BlockSpec((1, 1), full),
            pl.BlockSpec((HID + F_IN, C_OUT), full),
            pl.BlockSpec((1, C_OUT), full),
        ],
        out_specs=pl.BlockSpec((blk, C_OUT), lambda i: (i, 0)),
        out_shape=jax.ShapeDtypeStruct((N, C_OUT), jnp.float32),
    )(p, xs, dis, x, W1, b1, gamma, beta, attn_W, attn_b, fc_W, fc_b)


def kernel(x, edge_index, W1, b1, gamma, beta, attn_W, attn_b, fc_W, fc_b):
    ei = edge_index.astype(jnp.int32)
    npad = EPAD - ei.shape[1]
    ar = np.arange(npad, dtype=np.int32)
    pad = jnp.asarray(np.stack([ar % N, N + ar % (NACC - N)]))
    ei_p = jnp.concatenate([ei, pad], axis=1)

    deg_part = _deg_partials(ei_p)
    deg = (deg_part[:DR] + deg_part[DR:]).reshape(DR * 128, 1)[:N]
    xs, dis = _scale(x, deg)
    p = _segsum(xs, ei_p)
    return _post(
        p, xs, dis, x, W1,
        b1.reshape(1, HID), gamma.reshape(1, HID), beta.reshape(1, HID),
        attn_W, attn_b.reshape(1, 1), fc_W, fc_b.reshape(1, C_OUT))
